# baseline (speedup 1.0000x reference)
import functools

import jax
import jax.numpy as jnp
from jax import lax
from jax.experimental import pallas as pl
from jax.experimental.pallas import tpu as pltpu
from jax.experimental.pallas import tpu_sc as plsc

B = 16384
NF = 16
_NC = 2
_NS = 16
_NW = _NC * _NS
_BPW = B // _NW

def _sc_gather_body(ci_hbm, ri_hbm, ce_tab, re_tab, cb_tab, rb_tab,
                    ce_out, re_out, cb_out, rb_out,
                    ci_v, ri_v, ce_v, re_v, cb_v, rb_v, sem):
    wid = lax.axis_index("s") * _NC + lax.axis_index("c")
    base = wid * _BPW
    pltpu.sync_copy(ci_hbm.at[pl.ds(base, _BPW)], ci_v)
    pltpu.sync_copy(ri_hbm.at[pl.ds(base, _BPW)], ri_v)
    c1 = pltpu.async_copy(ce_tab.at[ci_v], ce_v, sem)
    c2 = pltpu.async_copy(re_tab.at[ri_v], re_v, sem)
    c3 = pltpu.async_copy(cb_tab.at[ci_v], cb_v, sem)
    c4 = pltpu.async_copy(rb_tab.at[ri_v], rb_v, sem)
    c1.wait()
    c2.wait()
    c3.wait()
    c4.wait()
    pltpu.sync_copy(ce_v, ce_out.at[pl.ds(base, _BPW)])
    pltpu.sync_copy(re_v, re_out.at[pl.ds(base, _BPW)])
    pltpu.sync_copy(cb_v, cb_out.at[pl.ds(base, _BPW)])
    pltpu.sync_copy(rb_v, rb_out.at[pl.ds(base, _BPW)])


@functools.cache
def _sc_gather_kernel():
    return functools.partial(
        pl.kernel,
        mesh=plsc.VectorSubcoreMesh(core_axis_name="c", subcore_axis_name="s"),
        out_type=[
            jax.ShapeDtypeStruct((B, NF), jnp.float32),
            jax.ShapeDtypeStruct((B, NF), jnp.float32),
            jax.ShapeDtypeStruct((B, 1), jnp.float32),
            jax.ShapeDtypeStruct((B, 1), jnp.float32),
        ],
        scratch_types=[
            pltpu.VMEM((_BPW,), jnp.int32),
            pltpu.VMEM((_BPW,), jnp.int32),
            pltpu.VMEM((_BPW, NF), jnp.float32),
            pltpu.VMEM((_BPW, NF), jnp.float32),
            pltpu.VMEM((_BPW, 1), jnp.float32),
            pltpu.VMEM((_BPW, 1), jnp.float32),
            pltpu.SemaphoreType.DMA,
        ],
    )(_sc_gather_body)


_BLK = 2048


def _tc_mlp_body(cf, rf, ce, re, cb, rb,
                 w1a, w1b, b1, w2, b2,
                 ew1h, ew1f, eb1, ew2, eb2, wo, bo, out):
    h = jnp.maximum(
        jnp.dot(cf[...], w1a[...], preferred_element_type=jnp.float32)
        + jnp.dot(rf[...], w1b[...], preferred_element_type=jnp.float32)
        + b1[...], 0.0)
    h2 = jnp.maximum(
        jnp.dot(h, w2[...], preferred_element_type=jnp.float32) + b2[...], 0.0)
    fm = ce[...] * re[...]
    e1 = jnp.maximum(
        jnp.dot(h2, ew1h[...], preferred_element_type=jnp.float32)
        + jnp.dot(fm, ew1f[...], preferred_element_type=jnp.float32)
        + eb1[...], 0.0)
    e2 = jnp.maximum(
        jnp.dot(e1, ew2[...], preferred_element_type=jnp.float32) + eb2[...], 0.0)
    eo = jnp.sum(e2 * wo[...], axis=1, keepdims=True) + bo[...]
    out[...] = cb[...] + rb[...] + eo


def _full(shape):
    return pl.BlockSpec(shape, lambda i: (0, 0))


def _tc_mlp(cf, rf, ce, re, cb, rb, w1a, w1b, b1, w2, b2,
            ew1h, ew1f, eb1, ew2, eb2, wo, bo, interpret=False):
    grid = (B // _BLK,)
    batch = lambda w: pl.BlockSpec((_BLK, w), lambda i: (i, 0))
    return pl.pallas_call(
        _tc_mlp_body,
        grid=grid,
        in_specs=[
            batch(64), batch(64), batch(NF), batch(NF), batch(1), batch(1),
            _full((64, 16)), _full((64, 16)), _full((1, 16)),
            _full((16, 8)), _full((1, 8)),
            _full((8, 8)), _full((NF, 8)), _full((1, 8)),
            _full((8, 4)), _full((1, 4)),
            _full((1, 4)), _full((1, 1)),
        ],
        out_specs=pl.BlockSpec((_BLK, 1), lambda i: (i, 0)),
        out_shape=jax.ShapeDtypeStruct((B, 1), jnp.float32),
        interpret=interpret,
    )(cf, rf, ce, re, cb, rb, w1a, w1b, b1, w2, b2,
      ew1h, ew1f, eb1, ew2, eb2, wo, bo)


def kernel(coil_indices, recipe_indices, coil_features, recipe_features,
           coil_emb, recipe_emb, coil_bias, recipe_bias,
           mlp_W1, mlp_b1, mlp_W2, mlp_b2,
           ens_W1, ens_b1, ens_W2, ens_b2, ens_Wo, ens_bo):
    ci = coil_indices.astype(jnp.int32)
    ri = recipe_indices.astype(jnp.int32)
    ce = jnp.take(coil_emb, ci, axis=0)
    re = jnp.take(recipe_emb, ri, axis=0)
    cb = jnp.take(coil_bias, ci, axis=0)
    rb = jnp.take(recipe_bias, ri, axis=0)
    pred = _tc_mlp(
        coil_features, recipe_features, ce, re, cb, rb,
        mlp_W1[:64], mlp_W1[64:], mlp_b1.reshape(1, 16),
        mlp_W2, mlp_b2.reshape(1, 8),
        ens_W1[:8], ens_W1[8:], ens_b1.reshape(1, 8),
        ens_W2, ens_b2.reshape(1, 4),
        ens_Wo.reshape(1, 4), ens_bo.reshape(1, 1))
    return pred[:, 0]
